# baseline (speedup 1.0000x reference)
import functools

import jax
import jax.numpy as jnp
from jax import lax
from jax.experimental import pallas as pl
from jax.experimental.pallas import tpu as pltpu
from jax.experimental.pallas import tpu_sc as plsc

VOCAB = 100000
EMBED_DIM = 128
CHUNK = 128


def _make_kernel(n_tokens: int):
    info = plsc.get_sparse_core_info()
    nw = info.num_cores * info.num_subcores
    assert n_tokens % (nw * CHUNK) == 0
    chunks_per_w = n_tokens // (nw * CHUNK)

    mesh = plsc.VectorSubcoreMesh(core_axis_name="c", subcore_axis_name="s")

    nbuf = 5
    assert chunks_per_w % nbuf == 0

    @functools.partial(
        pl.kernel,
        mesh=mesh,
        out_type=jax.ShapeDtypeStruct((n_tokens, EMBED_DIM), jnp.float32),
        scratch_types=[
            pltpu.VMEM((chunks_per_w, CHUNK), jnp.int32),
            pltpu.VMEM((nbuf, CHUNK, EMBED_DIM), jnp.float32),
            pltpu.SemaphoreType.DMA((nbuf,)),
            pltpu.SemaphoreType.DMA((nbuf,)),
        ],
    )
    def emb_kernel(idx_hbm, table_hbm, out_hbm, idx_v, rows_v, gsem, wsem):
        nc = info.num_cores
        wid = lax.axis_index("s") * nc + lax.axis_index("c")
        base = wid * (chunks_per_w * CHUNK)
        pltpu.sync_copy(idx_hbm.at[wid], idx_v)

        def gather_start(j, b):
            pltpu.async_copy(table_hbm.at[idx_v.at[j]], rows_v.at[b], gsem.at[b])

        def gather_wait(b):
            pltpu.make_async_copy(
                table_hbm.at[idx_v.at[0]], rows_v.at[b], gsem.at[b]
            ).wait()

        for b in range(nbuf):
            gather_start(b, b)

        n_outer = chunks_per_w // nbuf

        def write_wait(b):
            pltpu.make_async_copy(
                rows_v.at[b], out_hbm.at[pl.ds(base, CHUNK)], wsem.at[b]
            ).wait()

        def step(g, carry):
            for b in range(nbuf):
                j = g * nbuf + b
                gather_wait(b)
                pltpu.async_copy(
                    rows_v.at[b], out_hbm.at[pl.ds(base + j * CHUNK, CHUNK)], wsem.at[b]
                )
            for b in range(nbuf):

                @pl.when(g < n_outer - 1)
                def _():
                    write_wait(b)
                    gather_start((g + 1) * nbuf + b, b)

            return carry

        lax.fori_loop(0, n_outer, step, 0)
        for b in range(nbuf):
            write_wait(b)

    return emb_kernel, nw, chunks_per_w


def kernel(token_ids, table):
    b, t = token_ids.shape
    n_tokens = b * t
    emb_kernel, nw, chunks_per_w = _make_kernel(n_tokens)
    idx = token_ids.astype(jnp.int32).reshape(nw, chunks_per_w, CHUNK)
    out = emb_kernel(idx, table)
    return out.reshape(b, t, EMBED_DIM)


# speedup vs baseline: 1.0376x; 1.0376x over previous
import functools

import jax
import jax.numpy as jnp
from jax import lax
from jax.experimental import pallas as pl
from jax.experimental.pallas import tpu as pltpu
from jax.experimental.pallas import tpu_sc as plsc

VOCAB = 100000
EMBED_DIM = 128
CHUNK = 128


def _make_kernel(n_tokens: int):
    info = plsc.get_sparse_core_info()
    nw = info.num_cores * info.num_subcores
    assert n_tokens % (nw * CHUNK) == 0
    chunks_per_w = n_tokens // (nw * CHUNK)

    mesh = plsc.VectorSubcoreMesh(core_axis_name="c", subcore_axis_name="s")

    nbuf = 7
    assert chunks_per_w > nbuf

    @functools.partial(
        pl.kernel,
        mesh=mesh,
        out_type=jax.ShapeDtypeStruct((n_tokens, EMBED_DIM), jnp.float32),
        scratch_types=[
            pltpu.VMEM((chunks_per_w, CHUNK), jnp.int32),
            pltpu.VMEM((nbuf, CHUNK, EMBED_DIM), jnp.float32),
            pltpu.SemaphoreType.DMA((nbuf,)),
            pltpu.SemaphoreType.DMA((nbuf,)),
        ],
    )
    def emb_kernel(idx_hbm, table_hbm, out_hbm, idx_v, rows_v, gsem, wsem):
        nc = info.num_cores
        wid = lax.axis_index("s") * nc + lax.axis_index("c")
        base = wid * (chunks_per_w * CHUNK)
        pltpu.sync_copy(idx_hbm.at[wid], idx_v)

        def gather_start(j, b):
            pltpu.async_copy(table_hbm.at[idx_v.at[j]], rows_v.at[b], gsem.at[b])

        def gather_wait(b):
            pltpu.make_async_copy(
                table_hbm.at[idx_v.at[0]], rows_v.at[b], gsem.at[b]
            ).wait()

        for b in range(nbuf):
            gather_start(b, b)

        def write_wait(b):
            pltpu.make_async_copy(
                rows_v.at[b], out_hbm.at[pl.ds(base, CHUNK)], wsem.at[b]
            ).wait()

        def step(j, carry):
            b = lax.rem(j, nbuf)
            gather_wait(b)
            pltpu.async_copy(
                rows_v.at[b], out_hbm.at[pl.ds(base + j * CHUNK, CHUNK)], wsem.at[b]
            )

            @pl.when(j + nbuf < chunks_per_w)
            def _():
                write_wait(b)
                gather_start(j + nbuf, b)

            return carry

        lax.fori_loop(0, chunks_per_w, step, 0)
        for b in range(nbuf):
            write_wait(b)

    return emb_kernel, nw, chunks_per_w


def kernel(token_ids, table):
    b, t = token_ids.shape
    n_tokens = b * t
    emb_kernel, nw, chunks_per_w = _make_kernel(n_tokens)
    idx = token_ids.astype(jnp.int32).reshape(nw, chunks_per_w, CHUNK)
    out = emb_kernel(idx, table)
    return out.reshape(b, t, EMBED_DIM)
